# Initial kernel scaffold; baseline (speedup 1.0000x reference)
import jax
import jax.numpy as jnp
from jax.experimental import pallas as pl


def kernel(x, edge_index, W1, b1, W2, b2, W3, b3):
    raise NotImplementedError("write your pallas kernel here")


# speedup vs baseline: 24.7703x; 24.7703x over previous
import functools

import jax
import jax.numpy as jnp
from jax import lax
from jax.experimental import pallas as pl
from jax.experimental.pallas import tpu as pltpu
from jax.experimental.pallas import tpu_sc as plsc

N = 10000
E = 320000
D = 128
H1 = 32
H2 = 16

NC = 2
NS = 16
NW = NC * NS
CH = 128
CHUNKS = -(-E // (NW * CH))
EPAD = NW * CHUNKS * CH
NP = -(-N // 128) * 128
RPT = NP // NS

_mesh = plsc.VectorSubcoreMesh(core_axis_name="c", subcore_axis_name="s")
_sc_params = pltpu.CompilerParams(use_tc_tiling_on_sc=False)


@functools.partial(
    pl.kernel,
    out_type=jax.ShapeDtypeStruct((NC, NP, 16), jnp.float32),
    mesh=_mesh,
    scratch_types=[
        pltpu.VMEM_SHARED((NP, 16), jnp.float32),
        pltpu.VMEM((CHUNKS, CH), jnp.int32),
        pltpu.VMEM((CH, 16), jnp.float32),
    ],
    compiler_params=_sc_params,
)
def _sc_degree(dst_hbm, zeros_hbm, ones_hbm, out_hbm, acc_sh, didx, ones_v):
    c = lax.axis_index("c")
    s = lax.axis_index("s")
    wid = s * NC + c
    r0 = s * RPT
    pltpu.sync_copy(zeros_hbm.at[pl.ds(r0, RPT)], acc_sh.at[pl.ds(r0, RPT)])
    pltpu.sync_copy(dst_hbm.at[wid], didx)
    pltpu.sync_copy(ones_hbm, ones_v)
    plsc.subcore_barrier()

    def chunk(j, carry):
        pltpu.sync_copy(ones_v, acc_sh.at[didx.at[j]], add=True)
        return carry

    lax.fori_loop(0, CHUNKS, chunk, 0)
    plsc.subcore_barrier()
    pltpu.sync_copy(acc_sh.at[pl.ds(r0, RPT)], out_hbm.at[c, pl.ds(r0, RPT)])


@functools.partial(
    pl.kernel,
    out_type=jax.ShapeDtypeStruct((NC, NP, H1), jnp.float32),
    mesh=_mesh,
    scratch_types=[
        pltpu.VMEM_SHARED((NP, H1), jnp.float32),
        pltpu.VMEM((CHUNKS, CH), jnp.int32),
        pltpu.VMEM((CHUNKS, CH), jnp.int32),
        pltpu.VMEM((CH, H1), jnp.float32),
        pltpu.SemaphoreType.DMA,
    ],
    compiler_params=_sc_params,
)
def _sc_push(src_hbm, dst_hbm, h_hbm, zeros_hbm, out_hbm,
             acc_sh, sidx, didx, rows, sem):
    c = lax.axis_index("c")
    s = lax.axis_index("s")
    wid = s * NC + c
    r0 = s * RPT
    pltpu.sync_copy(zeros_hbm.at[pl.ds(r0, RPT)], acc_sh.at[pl.ds(r0, RPT)])
    pltpu.sync_copy(src_hbm.at[wid], sidx)
    pltpu.sync_copy(dst_hbm.at[wid], didx)
    plsc.subcore_barrier()

    def chunk(j, carry):
        pltpu.async_copy(h_hbm.at[sidx.at[j]], rows, sem).wait()
        pltpu.sync_copy(rows, acc_sh.at[didx.at[j]], add=True)
        return carry

    lax.fori_loop(0, CHUNKS, chunk, 0)
    plsc.subcore_barrier()
    pltpu.sync_copy(acc_sh.at[pl.ds(r0, RPT)], out_hbm.at[c, pl.ds(r0, RPT)])


_BM = 2000


def _tc1_body(x_ref, d0_ref, d1_ref, w1_ref, h1s_ref, dinv_ref):
    deg = 1.0 + d0_ref[:, :1] + d1_ref[:, :1]
    dinv = lax.rsqrt(deg)
    h = jnp.dot(x_ref[...], w1_ref[...], preferred_element_type=jnp.float32)
    h1s_ref[...] = h * dinv
    dinv_ref[...] = dinv


def _tc1(x, d0, d1, W1):
    return pl.pallas_call(
        _tc1_body,
        grid=(N // _BM,),
        in_specs=[
            pl.BlockSpec((_BM, D), lambda i: (i, 0)),
            pl.BlockSpec((_BM, 16), lambda i: (i, 0)),
            pl.BlockSpec((_BM, 16), lambda i: (i, 0)),
            pl.BlockSpec((D, H1), lambda i: (0, 0)),
        ],
        out_specs=[
            pl.BlockSpec((_BM, H1), lambda i: (i, 0)),
            pl.BlockSpec((_BM, 1), lambda i: (i, 0)),
        ],
        out_shape=[
            jax.ShapeDtypeStruct((N, H1), jnp.float32),
            jax.ShapeDtypeStruct((N, 1), jnp.float32),
        ],
    )(x, d0, d1, W1)


def _tc2_body(p0_ref, p1_ref, h1s_ref, dinv_ref, w23_ref, b1_ref, gs_ref):
    dinv = dinv_ref[...]
    hidden1 = dinv * (p0_ref[...] + p1_ref[...] + h1s_ref[...]) + b1_ref[...]
    g = jnp.dot(hidden1, w23_ref[...], preferred_element_type=jnp.float32)
    gs_ref[...] = dinv * g


def _tc2(p0, p1, h1s, dinv, W23, b1r):
    return pl.pallas_call(
        _tc2_body,
        grid=(N // _BM,),
        in_specs=[
            pl.BlockSpec((_BM, H1), lambda i: (i, 0)),
            pl.BlockSpec((_BM, H1), lambda i: (i, 0)),
            pl.BlockSpec((_BM, H1), lambda i: (i, 0)),
            pl.BlockSpec((_BM, 1), lambda i: (i, 0)),
            pl.BlockSpec((H1, 2 * H2), lambda i: (0, 0)),
            pl.BlockSpec((1, H1), lambda i: (0, 0)),
        ],
        out_specs=pl.BlockSpec((_BM, 2 * H2), lambda i: (i, 0)),
        out_shape=jax.ShapeDtypeStruct((N, 2 * H2), jnp.float32),
    )(p0, p1, h1s, dinv, W23, b1r)


def _tc3_body(q0_ref, q1_ref, gs_ref, dinv_ref, b23_ref, mu_ref, lv_ref):
    m = dinv_ref[...] * (q0_ref[...] + q1_ref[...] + gs_ref[...]) + b23_ref[...]
    mu_ref[...] = m[:, :H2]
    lv_ref[...] = m[:, H2:]


def _tc3(q0, q1, gs, dinv, b23r):
    return pl.pallas_call(
        _tc3_body,
        grid=(N // _BM,),
        in_specs=[
            pl.BlockSpec((_BM, 2 * H2), lambda i: (i, 0)),
            pl.BlockSpec((_BM, 2 * H2), lambda i: (i, 0)),
            pl.BlockSpec((_BM, 2 * H2), lambda i: (i, 0)),
            pl.BlockSpec((_BM, 1), lambda i: (i, 0)),
            pl.BlockSpec((1, 2 * H2), lambda i: (0, 0)),
        ],
        out_specs=[
            pl.BlockSpec((_BM, H2), lambda i: (i, 0)),
            pl.BlockSpec((_BM, H2), lambda i: (i, 0)),
        ],
        out_shape=[
            jax.ShapeDtypeStruct((N, H2), jnp.float32),
            jax.ShapeDtypeStruct((N, H2), jnp.float32),
        ],
    )(q0, q1, gs, dinv, b23r)


_BMA = 200


def _adj_body(a_ref, bt_ref, out_ref):
    z = jnp.dot(a_ref[...], bt_ref[...], preferred_element_type=jnp.float32)
    out_ref[...] = jax.nn.sigmoid(z)


def _tc_adj(mu, muT):
    return pl.pallas_call(
        _adj_body,
        grid=(N // _BMA,),
        in_specs=[
            pl.BlockSpec((_BMA, H2), lambda i: (i, 0)),
            pl.BlockSpec((H2, N), lambda i: (0, 0)),
        ],
        out_specs=pl.BlockSpec((_BMA, N), lambda i: (i, 0)),
        out_shape=jax.ShapeDtypeStruct((N, N), jnp.float32),
    )(mu, muT)


def kernel(x, edge_index, W1, b1, W2, b2, W3, b3):
    src = edge_index[0]
    dst = edge_index[1]
    pad = EPAD - E
    pad_idx = (N + (jnp.arange(pad, dtype=jnp.int32) % 16)).astype(jnp.int32)
    src_t = jnp.concatenate([src, pad_idx]).reshape(NW, CHUNKS, CH)
    dst_t = jnp.concatenate([dst, pad_idx]).reshape(NW, CHUNKS, CH)

    zeros16 = jnp.zeros((NP, 16), jnp.float32)
    zeros32 = jnp.zeros((NP, H1), jnp.float32)
    ones_blk = jnp.ones((CH, 16), jnp.float32)

    degp = _sc_degree(dst_t, zeros16, ones_blk)
    h1s, dinv = _tc1(x, degp[0, :N], degp[1, :N], W1)

    h1s_p = jnp.pad(h1s, ((0, NP - N), (0, 0)))
    agg1 = _sc_push(src_t, dst_t, h1s_p, zeros32)

    W23 = jnp.concatenate([W2, W3], axis=1)
    gs = _tc2(agg1[0, :N], agg1[1, :N], h1s, dinv, W23, b1.reshape(1, H1))

    gs_p = jnp.pad(gs, ((0, NP - N), (0, 0)))
    agg2 = _sc_push(src_t, dst_t, gs_p, zeros32)

    b23 = jnp.concatenate([b2, b3]).reshape(1, 2 * H2)
    mu, logvar = _tc3(agg2[0, :N], agg2[1, :N], gs, dinv, b23)

    adj = _tc_adj(mu, mu.T)
    return (adj, mu, logvar)


# speedup vs baseline: 33.0155x; 1.3329x over previous
import functools

import jax
import jax.numpy as jnp
from jax import lax
from jax.experimental import pallas as pl
from jax.experimental.pallas import tpu as pltpu
from jax.experimental.pallas import tpu_sc as plsc

N = 10000
E = 320000
D = 128
H1 = 32
H2 = 16

NC = 2
NS = 16
NW = NC * NS
CH = 128
NBUF = 4
CHUNKS = -(-(-(-E // (NW * CH))) // NBUF) * NBUF
EPAD = NW * CHUNKS * CH
NP = -(-N // 128) * 128
RPT = NP // NS

_mesh = plsc.VectorSubcoreMesh(core_axis_name="c", subcore_axis_name="s")
_sc_params = pltpu.CompilerParams(use_tc_tiling_on_sc=False)


_DGRP = 8


@functools.partial(
    pl.kernel,
    out_type=jax.ShapeDtypeStruct((NC, NP, 16), jnp.float32),
    mesh=_mesh,
    scratch_types=[
        pltpu.VMEM_SHARED((NP, 16), jnp.float32),
        pltpu.VMEM((CHUNKS, CH), jnp.int32),
        pltpu.VMEM((CH, 16), jnp.float32),
        pltpu.SemaphoreType.DMA,
    ],
    compiler_params=_sc_params,
)
def _sc_degree(dst_hbm, zeros_hbm, ones_hbm, out_hbm, acc_sh, didx, ones_v, sem):
    c = lax.axis_index("c")
    s = lax.axis_index("s")
    wid = s * NC + c
    r0 = s * RPT
    pltpu.sync_copy(zeros_hbm.at[pl.ds(r0, RPT)], acc_sh.at[pl.ds(r0, RPT)])
    pltpu.sync_copy(dst_hbm.at[wid], didx)
    pltpu.sync_copy(ones_hbm, ones_v)
    plsc.subcore_barrier()

    def wave(g, carry):
        for b in range(_DGRP):
            pltpu.async_copy(ones_v, acc_sh.at[didx.at[g * _DGRP + b]], sem,
                             add=True)
        for b in range(_DGRP):
            pltpu.make_async_copy(ones_v, acc_sh.at[didx.at[g * _DGRP + b]],
                                  sem).wait()
        return carry

    lax.fori_loop(0, CHUNKS // _DGRP, wave, 0)
    plsc.subcore_barrier()
    pltpu.sync_copy(acc_sh.at[pl.ds(r0, RPT)], out_hbm.at[c, pl.ds(r0, RPT)])


@functools.partial(
    pl.kernel,
    out_type=jax.ShapeDtypeStruct((NC, NP, H1), jnp.float32),
    mesh=_mesh,
    scratch_types=[
        pltpu.VMEM_SHARED((NP, H1), jnp.float32),
        pltpu.VMEM((CHUNKS, CH), jnp.int32),
        pltpu.VMEM((CHUNKS, CH), jnp.int32),
        *[pltpu.VMEM((CH, H1), jnp.float32) for _ in range(NBUF)],
        *[pltpu.SemaphoreType.DMA for _ in range(2 * NBUF)],
    ],
    compiler_params=_sc_params,
)
def _sc_push(src_hbm, dst_hbm, h_hbm, zeros_hbm, out_hbm,
             acc_sh, sidx, didx, *bufs_and_sems):
    rows = bufs_and_sems[:NBUF]
    gsem = bufs_and_sems[NBUF:2 * NBUF]
    ssem = bufs_and_sems[2 * NBUF:]
    c = lax.axis_index("c")
    s = lax.axis_index("s")
    wid = s * NC + c
    r0 = s * RPT
    pltpu.sync_copy(zeros_hbm.at[pl.ds(r0, RPT)], acc_sh.at[pl.ds(r0, RPT)])
    pltpu.sync_copy(src_hbm.at[wid], sidx)
    pltpu.sync_copy(dst_hbm.at[wid], didx)
    plsc.subcore_barrier()

    for b in range(NBUF):
        pltpu.async_copy(h_hbm.at[sidx.at[b]], rows[b], gsem[b])

    def group(g, carry):
        for b in range(NBUF):
            j = g * NBUF + b
            pltpu.make_async_copy(h_hbm.at[sidx.at[j]], rows[b], gsem[b]).wait()
            pltpu.async_copy(rows[b], acc_sh.at[didx.at[j]], ssem[b], add=True)
            jn = j + NBUF

            @pl.when(jn < CHUNKS)
            def _():
                pltpu.make_async_copy(rows[b], acc_sh.at[didx.at[j]],
                                      ssem[b]).wait()
                pltpu.async_copy(h_hbm.at[sidx.at[jn]], rows[b], gsem[b])

        return carry

    lax.fori_loop(0, CHUNKS // NBUF, group, 0)
    for b in range(NBUF):
        pltpu.make_async_copy(rows[b], acc_sh.at[didx.at[0]], ssem[b]).wait()
    plsc.subcore_barrier()
    pltpu.sync_copy(acc_sh.at[pl.ds(r0, RPT)], out_hbm.at[c, pl.ds(r0, RPT)])


_BM = 2000


def _tc1_body(x_ref, d0_ref, d1_ref, w1_ref, h1s_ref, dinv_ref):
    deg = 1.0 + d0_ref[:, :1] + d1_ref[:, :1]
    dinv = lax.rsqrt(deg)
    h = jnp.dot(x_ref[...], w1_ref[...], preferred_element_type=jnp.float32)
    h1s_ref[...] = h * dinv
    dinv_ref[...] = dinv


def _tc1(x, d0, d1, W1):
    return pl.pallas_call(
        _tc1_body,
        grid=(N // _BM,),
        in_specs=[
            pl.BlockSpec((_BM, D), lambda i: (i, 0)),
            pl.BlockSpec((_BM, 16), lambda i: (i, 0)),
            pl.BlockSpec((_BM, 16), lambda i: (i, 0)),
            pl.BlockSpec((D, H1), lambda i: (0, 0)),
        ],
        out_specs=[
            pl.BlockSpec((_BM, H1), lambda i: (i, 0)),
            pl.BlockSpec((_BM, 1), lambda i: (i, 0)),
        ],
        out_shape=[
            jax.ShapeDtypeStruct((N, H1), jnp.float32),
            jax.ShapeDtypeStruct((N, 1), jnp.float32),
        ],
    )(x, d0, d1, W1)


def _tc2_body(p0_ref, p1_ref, h1s_ref, dinv_ref, w23_ref, b1_ref, gs_ref):
    dinv = dinv_ref[...]
    hidden1 = dinv * (p0_ref[...] + p1_ref[...] + h1s_ref[...]) + b1_ref[...]
    g = jnp.dot(hidden1, w23_ref[...], preferred_element_type=jnp.float32)
    gs_ref[...] = dinv * g


def _tc2(p0, p1, h1s, dinv, W23, b1r):
    return pl.pallas_call(
        _tc2_body,
        grid=(N // _BM,),
        in_specs=[
            pl.BlockSpec((_BM, H1), lambda i: (i, 0)),
            pl.BlockSpec((_BM, H1), lambda i: (i, 0)),
            pl.BlockSpec((_BM, H1), lambda i: (i, 0)),
            pl.BlockSpec((_BM, 1), lambda i: (i, 0)),
            pl.BlockSpec((H1, 2 * H2), lambda i: (0, 0)),
            pl.BlockSpec((1, H1), lambda i: (0, 0)),
        ],
        out_specs=pl.BlockSpec((_BM, 2 * H2), lambda i: (i, 0)),
        out_shape=jax.ShapeDtypeStruct((N, 2 * H2), jnp.float32),
    )(p0, p1, h1s, dinv, W23, b1r)


def _tc3_body(q0_ref, q1_ref, gs_ref, dinv_ref, b23_ref, mu_ref, lv_ref):
    m = dinv_ref[...] * (q0_ref[...] + q1_ref[...] + gs_ref[...]) + b23_ref[...]
    mu_ref[...] = m[:, :H2]
    lv_ref[...] = m[:, H2:]


def _tc3(q0, q1, gs, dinv, b23r):
    return pl.pallas_call(
        _tc3_body,
        grid=(N // _BM,),
        in_specs=[
            pl.BlockSpec((_BM, 2 * H2), lambda i: (i, 0)),
            pl.BlockSpec((_BM, 2 * H2), lambda i: (i, 0)),
            pl.BlockSpec((_BM, 2 * H2), lambda i: (i, 0)),
            pl.BlockSpec((_BM, 1), lambda i: (i, 0)),
            pl.BlockSpec((1, 2 * H2), lambda i: (0, 0)),
        ],
        out_specs=[
            pl.BlockSpec((_BM, H2), lambda i: (i, 0)),
            pl.BlockSpec((_BM, H2), lambda i: (i, 0)),
        ],
        out_shape=[
            jax.ShapeDtypeStruct((N, H2), jnp.float32),
            jax.ShapeDtypeStruct((N, H2), jnp.float32),
        ],
    )(q0, q1, gs, dinv, b23r)


_BMA = 200


def _adj_body(a_ref, bt_ref, out_ref):
    z = jnp.dot(a_ref[...], bt_ref[...], preferred_element_type=jnp.float32)
    out_ref[...] = jax.nn.sigmoid(z)


def _tc_adj(mu, muT):
    return pl.pallas_call(
        _adj_body,
        grid=(N // _BMA,),
        in_specs=[
            pl.BlockSpec((_BMA, H2), lambda i: (i, 0)),
            pl.BlockSpec((H2, N), lambda i: (0, 0)),
        ],
        out_specs=pl.BlockSpec((_BMA, N), lambda i: (i, 0)),
        out_shape=jax.ShapeDtypeStruct((N, N), jnp.float32),
    )(mu, muT)


def kernel(x, edge_index, W1, b1, W2, b2, W3, b3):
    src = edge_index[0]
    dst = edge_index[1]
    pad = EPAD - E
    pad_idx = (N + (jnp.arange(pad, dtype=jnp.int32) % (NP - N))).astype(jnp.int32)
    src_t = jnp.concatenate([src, pad_idx]).reshape(NW, CHUNKS, CH)
    dst_t = jnp.concatenate([dst, pad_idx]).reshape(NW, CHUNKS, CH)

    zeros16 = jnp.zeros((NP, 16), jnp.float32)
    zeros32 = jnp.zeros((NP, H1), jnp.float32)
    ones_blk = jnp.ones((CH, 16), jnp.float32)

    degp = _sc_degree(dst_t, zeros16, ones_blk)
    h1s, dinv = _tc1(x, degp[0, :N], degp[1, :N], W1)

    h1s_p = jnp.pad(h1s, ((0, NP - N), (0, 0)))
    agg1 = _sc_push(src_t, dst_t, h1s_p, zeros32)

    W23 = jnp.concatenate([W2, W3], axis=1)
    gs = _tc2(agg1[0, :N], agg1[1, :N], h1s, dinv, W23, b1.reshape(1, H1))

    gs_p = jnp.pad(gs, ((0, NP - N), (0, 0)))
    agg2 = _sc_push(src_t, dst_t, gs_p, zeros32)

    b23 = jnp.concatenate([b2, b3]).reshape(1, 2 * H2)
    mu, logvar = _tc3(agg2[0, :N], agg2[1, :N], gs, dinv, b23)

    adj = _tc_adj(mu, mu.T)
    return (adj, mu, logvar)


# speedup vs baseline: 35.3627x; 1.0711x over previous
import functools

import jax
import jax.numpy as jnp
from jax import lax
from jax.experimental import pallas as pl
from jax.experimental.pallas import tpu as pltpu
from jax.experimental.pallas import tpu_sc as plsc

N = 10000
E = 320000
D = 128
H1 = 32
H2 = 16

NC = 2
NS = 16
NW = NC * NS
CH = 128
NBUF = 4
CHUNKS = -(-(-(-E // (NW * CH))) // NBUF) * NBUF
EPAD = NW * CHUNKS * CH
NP = -(-N // 128) * 128
RPT = NP // NS

_mesh = plsc.VectorSubcoreMesh(core_axis_name="c", subcore_axis_name="s")
_sc_params = pltpu.CompilerParams(use_tc_tiling_on_sc=False)


_DGRP = 8


@functools.partial(
    pl.kernel,
    out_type=jax.ShapeDtypeStruct((NC, NP, 16), jnp.float32),
    mesh=_mesh,
    scratch_types=[
        pltpu.VMEM_SHARED((NP, 16), jnp.float32),
        pltpu.VMEM((CHUNKS, CH), jnp.int32),
        pltpu.VMEM((CH, 16), jnp.float32),
        pltpu.SemaphoreType.DMA,
    ],
    compiler_params=_sc_params,
)
def _sc_degree(dst_hbm, zeros_hbm, ones_hbm, out_hbm, acc_sh, didx, ones_v, sem):
    c = lax.axis_index("c")
    s = lax.axis_index("s")
    wid = s * NC + c
    r0 = s * RPT
    pltpu.sync_copy(zeros_hbm.at[pl.ds(r0, RPT)], acc_sh.at[pl.ds(r0, RPT)])
    pltpu.sync_copy(dst_hbm.at[wid], didx)
    pltpu.sync_copy(ones_hbm, ones_v)
    plsc.subcore_barrier()

    def wave(g, carry):
        for b in range(_DGRP):
            pltpu.async_copy(ones_v, acc_sh.at[didx.at[g * _DGRP + b]], sem,
                             add=True)
        for b in range(_DGRP):
            pltpu.make_async_copy(ones_v, acc_sh.at[didx.at[g * _DGRP + b]],
                                  sem).wait()
        return carry

    lax.fori_loop(0, CHUNKS // _DGRP, wave, 0)
    plsc.subcore_barrier()
    pltpu.sync_copy(acc_sh.at[pl.ds(r0, RPT)], out_hbm.at[c, pl.ds(r0, RPT)])


@functools.partial(
    pl.kernel,
    out_type=jax.ShapeDtypeStruct((NC, NP, H1), jnp.float32),
    mesh=_mesh,
    scratch_types=[
        pltpu.VMEM_SHARED((NP, H1), jnp.float32),
        pltpu.VMEM((CHUNKS, CH), jnp.int32),
        pltpu.VMEM((CHUNKS, CH), jnp.int32),
        *[pltpu.VMEM((CH, H1), jnp.float32) for _ in range(NBUF)],
        *[pltpu.SemaphoreType.DMA for _ in range(2 * NBUF)],
    ],
    compiler_params=_sc_params,
)
def _sc_push(src_hbm, dst_hbm, h_hbm, zeros_hbm, out_hbm,
             acc_sh, sidx, didx, *bufs_and_sems):
    rows = bufs_and_sems[:NBUF]
    gsem = bufs_and_sems[NBUF:2 * NBUF]
    ssem = bufs_and_sems[2 * NBUF:]
    c = lax.axis_index("c")
    s = lax.axis_index("s")
    wid = s * NC + c
    r0 = s * RPT
    pltpu.sync_copy(zeros_hbm.at[pl.ds(r0, RPT)], acc_sh.at[pl.ds(r0, RPT)])
    pltpu.sync_copy(src_hbm.at[wid], sidx)
    pltpu.sync_copy(dst_hbm.at[wid], didx)
    plsc.subcore_barrier()

    for b in range(NBUF):
        pltpu.async_copy(h_hbm.at[sidx.at[b]], rows[b], gsem[b])

    def group(g, carry):
        for b in range(NBUF):
            j = g * NBUF + b
            pltpu.make_async_copy(h_hbm.at[sidx.at[j]], rows[b], gsem[b]).wait()
            pltpu.async_copy(rows[b], acc_sh.at[didx.at[j]], ssem[b], add=True)
            jn = j + NBUF

            @pl.when(jn < CHUNKS)
            def _():
                pltpu.make_async_copy(rows[b], acc_sh.at[didx.at[j]],
                                      ssem[b]).wait()
                pltpu.async_copy(h_hbm.at[sidx.at[jn]], rows[b], gsem[b])

        return carry

    lax.fori_loop(0, CHUNKS // NBUF, group, 0)
    for b in range(NBUF):
        pltpu.make_async_copy(rows[b], acc_sh.at[didx.at[0]], ssem[b]).wait()
    plsc.subcore_barrier()
    pltpu.sync_copy(acc_sh.at[pl.ds(r0, RPT)], out_hbm.at[c, pl.ds(r0, RPT)])


_BM = 2000


def _tc1_body(x_ref, d0_ref, d1_ref, w1_ref, h1s_ref, dinv_ref):
    deg = 1.0 + d0_ref[:, :1] + d1_ref[:, :1]
    dinv = lax.rsqrt(deg)
    h = jnp.dot(x_ref[...], w1_ref[...], preferred_element_type=jnp.float32)
    h1s_ref[...] = h * dinv
    dinv_ref[...] = dinv


def _tc1(x, d0, d1, W1):
    return pl.pallas_call(
        _tc1_body,
        grid=(N // _BM,),
        in_specs=[
            pl.BlockSpec((_BM, D), lambda i: (i, 0)),
            pl.BlockSpec((_BM, 16), lambda i: (i, 0)),
            pl.BlockSpec((_BM, 16), lambda i: (i, 0)),
            pl.BlockSpec((D, H1), lambda i: (0, 0)),
        ],
        out_specs=[
            pl.BlockSpec((_BM, H1), lambda i: (i, 0)),
            pl.BlockSpec((_BM, 1), lambda i: (i, 0)),
        ],
        out_shape=[
            jax.ShapeDtypeStruct((N, H1), jnp.float32),
            jax.ShapeDtypeStruct((N, 1), jnp.float32),
        ],
    )(x, d0, d1, W1)


def _tc2_body(p0_ref, p1_ref, h1s_ref, dinv_ref, w23_ref, b1_ref, gs_ref):
    dinv = dinv_ref[...]
    hidden1 = dinv * (p0_ref[...] + p1_ref[...] + h1s_ref[...]) + b1_ref[...]
    g = jnp.dot(hidden1, w23_ref[...], preferred_element_type=jnp.float32)
    gs_ref[...] = dinv * g


def _tc2(p0, p1, h1s, dinv, W23, b1r):
    return pl.pallas_call(
        _tc2_body,
        grid=(N // _BM,),
        in_specs=[
            pl.BlockSpec((_BM, H1), lambda i: (i, 0)),
            pl.BlockSpec((_BM, H1), lambda i: (i, 0)),
            pl.BlockSpec((_BM, H1), lambda i: (i, 0)),
            pl.BlockSpec((_BM, 1), lambda i: (i, 0)),
            pl.BlockSpec((H1, 2 * H2), lambda i: (0, 0)),
            pl.BlockSpec((1, H1), lambda i: (0, 0)),
        ],
        out_specs=pl.BlockSpec((_BM, 2 * H2), lambda i: (i, 0)),
        out_shape=jax.ShapeDtypeStruct((N, 2 * H2), jnp.float32),
    )(p0, p1, h1s, dinv, W23, b1r)


def _tc3_body(q0_ref, q1_ref, gs_ref, dinv_ref, b23_ref, mu_ref, lv_ref):
    m = dinv_ref[...] * (q0_ref[...] + q1_ref[...] + gs_ref[...]) + b23_ref[...]
    mu_ref[...] = m[:, :H2]
    lv_ref[...] = m[:, H2:]


def _tc3(q0, q1, gs, dinv, b23r):
    return pl.pallas_call(
        _tc3_body,
        grid=(N // _BM,),
        in_specs=[
            pl.BlockSpec((_BM, 2 * H2), lambda i: (i, 0)),
            pl.BlockSpec((_BM, 2 * H2), lambda i: (i, 0)),
            pl.BlockSpec((_BM, 2 * H2), lambda i: (i, 0)),
            pl.BlockSpec((_BM, 1), lambda i: (i, 0)),
            pl.BlockSpec((1, 2 * H2), lambda i: (0, 0)),
        ],
        out_specs=[
            pl.BlockSpec((_BM, H2), lambda i: (i, 0)),
            pl.BlockSpec((_BM, H2), lambda i: (i, 0)),
        ],
        out_shape=[
            jax.ShapeDtypeStruct((N, H2), jnp.float32),
            jax.ShapeDtypeStruct((N, H2), jnp.float32),
        ],
    )(q0, q1, gs, dinv, b23r)


_BMA = 200


def _adj_body(a_ref, bt_ref, out_ref):
    z = jnp.dot(a_ref[...], bt_ref[...], preferred_element_type=jnp.float32)
    out_ref[...] = 0.5 * jnp.tanh(0.5 * z) + 0.5


def _tc_adj(mu, muT):
    return pl.pallas_call(
        _adj_body,
        grid=(N // _BMA,),
        in_specs=[
            pl.BlockSpec((_BMA, H2), lambda i: (i, 0)),
            pl.BlockSpec((H2, N), lambda i: (0, 0)),
        ],
        out_specs=pl.BlockSpec((_BMA, N), lambda i: (i, 0)),
        out_shape=jax.ShapeDtypeStruct((N, N), jnp.float32),
    )(mu, muT)


def kernel(x, edge_index, W1, b1, W2, b2, W3, b3):
    src = edge_index[0]
    dst = edge_index[1]
    pad = EPAD - E
    pad_idx = (N + (jnp.arange(pad, dtype=jnp.int32) % (NP - N))).astype(jnp.int32)
    src_t = jnp.concatenate([src, pad_idx]).reshape(NW, CHUNKS, CH)
    dst_t = jnp.concatenate([dst, pad_idx]).reshape(NW, CHUNKS, CH)

    zeros16 = jnp.zeros((NP, 16), jnp.float32)
    zeros32 = jnp.zeros((NP, H1), jnp.float32)
    ones_blk = jnp.ones((CH, 16), jnp.float32)

    degp = _sc_degree(dst_t, zeros16, ones_blk)
    h1s, dinv = _tc1(x, degp[0, :N], degp[1, :N], W1)

    h1s_p = jnp.pad(h1s, ((0, NP - N), (0, 0)))
    agg1 = _sc_push(src_t, dst_t, h1s_p, zeros32)

    W23 = jnp.concatenate([W2, W3], axis=1)
    gs = _tc2(agg1[0, :N], agg1[1, :N], h1s, dinv, W23, b1.reshape(1, H1))

    gs_p = jnp.pad(gs, ((0, NP - N), (0, 0)))
    agg2 = _sc_push(src_t, dst_t, gs_p, zeros32)

    b23 = jnp.concatenate([b2, b3]).reshape(1, 2 * H2)
    mu, logvar = _tc3(agg2[0, :N], agg2[1, :N], gs, dinv, b23)

    adj = _tc_adj(mu, mu.T)
    return (adj, mu, logvar)


# speedup vs baseline: 37.9001x; 1.0718x over previous
import functools

import jax
import jax.numpy as jnp
from jax import lax
from jax.experimental import pallas as pl
from jax.experimental.pallas import tpu as pltpu
from jax.experimental.pallas import tpu_sc as plsc

N = 10000
E = 320000
D = 128
H1 = 32
H2 = 16

NC = 2
NS = 16
NW = NC * NS
CH = 128
NBUF = 8
CHUNKS = -(-(-(-E // (NW * CH))) // NBUF) * NBUF
EPAD = NW * CHUNKS * CH
NP = -(-N // 128) * 128
RPT = NP // NS

_mesh = plsc.VectorSubcoreMesh(core_axis_name="c", subcore_axis_name="s")
_sc_params = pltpu.CompilerParams(use_tc_tiling_on_sc=False)


_DGRP = 16


@functools.partial(
    pl.kernel,
    out_type=jax.ShapeDtypeStruct((NC, NP, 16), jnp.float32),
    mesh=_mesh,
    scratch_types=[
        pltpu.VMEM_SHARED((NP, 16), jnp.float32),
        pltpu.VMEM((CHUNKS, CH), jnp.int32),
        pltpu.VMEM((CH, 16), jnp.float32),
        pltpu.SemaphoreType.DMA,
    ],
    compiler_params=_sc_params,
)
def _sc_degree(dst_hbm, zeros_hbm, ones_hbm, out_hbm, acc_sh, didx, ones_v, sem):
    c = lax.axis_index("c")
    s = lax.axis_index("s")
    wid = s * NC + c
    r0 = s * RPT
    pltpu.sync_copy(zeros_hbm.at[pl.ds(r0, RPT)], acc_sh.at[pl.ds(r0, RPT)])
    pltpu.sync_copy(dst_hbm.at[wid], didx)
    pltpu.sync_copy(ones_hbm, ones_v)
    plsc.subcore_barrier()

    def wave(g, carry):
        for b in range(_DGRP):
            pltpu.async_copy(ones_v, acc_sh.at[didx.at[g * _DGRP + b]], sem,
                             add=True)
        for b in range(_DGRP):
            pltpu.make_async_copy(ones_v, acc_sh.at[didx.at[g * _DGRP + b]],
                                  sem).wait()
        return carry

    lax.fori_loop(0, CHUNKS // _DGRP, wave, 0)
    plsc.subcore_barrier()
    pltpu.sync_copy(acc_sh.at[pl.ds(r0, RPT)], out_hbm.at[c, pl.ds(r0, RPT)])


@functools.partial(
    pl.kernel,
    out_type=jax.ShapeDtypeStruct((NC, NP, H1), jnp.float32),
    mesh=_mesh,
    scratch_types=[
        pltpu.VMEM_SHARED((NP, H1), jnp.float32),
        pltpu.VMEM((CHUNKS, CH), jnp.int32),
        pltpu.VMEM((CHUNKS, CH), jnp.int32),
        *[pltpu.VMEM((CH, H1), jnp.float32) for _ in range(NBUF)],
        *[pltpu.SemaphoreType.DMA for _ in range(2 * NBUF)],
    ],
    compiler_params=_sc_params,
)
def _sc_push(src_hbm, dst_hbm, h_hbm, zeros_hbm, out_hbm,
             acc_sh, sidx, didx, *bufs_and_sems):
    rows = bufs_and_sems[:NBUF]
    gsem = bufs_and_sems[NBUF:2 * NBUF]
    ssem = bufs_and_sems[2 * NBUF:]
    c = lax.axis_index("c")
    s = lax.axis_index("s")
    wid = s * NC + c
    r0 = s * RPT
    pltpu.sync_copy(zeros_hbm.at[pl.ds(r0, RPT)], acc_sh.at[pl.ds(r0, RPT)])
    pltpu.sync_copy(src_hbm.at[wid], sidx)
    pltpu.sync_copy(dst_hbm.at[wid], didx)
    plsc.subcore_barrier()

    for b in range(NBUF):
        pltpu.async_copy(h_hbm.at[sidx.at[b]], rows[b], gsem[b])

    def group(g, carry):
        for b in range(NBUF):
            j = g * NBUF + b
            pltpu.make_async_copy(h_hbm.at[sidx.at[j]], rows[b], gsem[b]).wait()
            pltpu.async_copy(rows[b], acc_sh.at[didx.at[j]], ssem[b], add=True)
            jn = j + NBUF

            @pl.when(jn < CHUNKS)
            def _():
                pltpu.make_async_copy(rows[b], acc_sh.at[didx.at[j]],
                                      ssem[b]).wait()
                pltpu.async_copy(h_hbm.at[sidx.at[jn]], rows[b], gsem[b])

        return carry

    lax.fori_loop(0, CHUNKS // NBUF, group, 0)
    for b in range(NBUF):
        pltpu.make_async_copy(rows[b], acc_sh.at[didx.at[0]], ssem[b]).wait()
    plsc.subcore_barrier()
    pltpu.sync_copy(acc_sh.at[pl.ds(r0, RPT)], out_hbm.at[c, pl.ds(r0, RPT)])


_BMP = NP // 8
_BM = 2000


def _mm1_body(x_ref, w1_ref, h1_ref):
    h1_ref[...] = jnp.dot(x_ref[...], w1_ref[...],
                          preferred_element_type=jnp.float32)


def _tc_mm1(x_p, W1):
    return pl.pallas_call(
        _mm1_body,
        grid=(NP // _BMP,),
        in_specs=[
            pl.BlockSpec((_BMP, D), lambda i: (i, 0)),
            pl.BlockSpec((D, H1), lambda i: (0, 0)),
        ],
        out_specs=pl.BlockSpec((_BMP, H1), lambda i: (i, 0)),
        out_shape=jax.ShapeDtypeStruct((NP, H1), jnp.float32),
    )(x_p, W1)


def _scale_body(d0_ref, d1_ref, h1_ref, h1s_ref, dinv_ref):
    deg = 1.0 + d0_ref[0, :, :1] + d1_ref[0, :, :1]
    dinv = lax.rsqrt(deg)
    h1s_ref[...] = h1_ref[...] * dinv
    dinv_ref[...] = dinv


def _tc_scale(degp, h1):
    return pl.pallas_call(
        _scale_body,
        grid=(NP // _BMP,),
        in_specs=[
            pl.BlockSpec((1, _BMP, 16), lambda i: (0, i, 0)),
            pl.BlockSpec((1, _BMP, 16), lambda i: (1, i, 0)),
            pl.BlockSpec((_BMP, H1), lambda i: (i, 0)),
        ],
        out_specs=[
            pl.BlockSpec((_BMP, H1), lambda i: (i, 0)),
            pl.BlockSpec((_BMP, 1), lambda i: (i, 0)),
        ],
        out_shape=[
            jax.ShapeDtypeStruct((NP, H1), jnp.float32),
            jax.ShapeDtypeStruct((NP, 1), jnp.float32),
        ],
    )(degp, degp, h1)


def _mid_body(p0_ref, p1_ref, h1s_ref, dinv_ref, w23_ref, b1_ref, gs_ref):
    dinv = dinv_ref[...]
    hidden1 = dinv * (p0_ref[0] + p1_ref[0] + h1s_ref[...]) + b1_ref[...]
    g = jnp.dot(hidden1, w23_ref[...], preferred_element_type=jnp.float32)
    i = pl.program_id(0)
    row = i * _BMP + lax.broadcasted_iota(jnp.int32, (_BMP, 2 * H2), 0)
    gs_ref[...] = jnp.where(row < N, dinv * g, 0.0)


def _tc_mid(agg1, h1s, dinv, W23, b1r):
    return pl.pallas_call(
        _mid_body,
        grid=(NP // _BMP,),
        in_specs=[
            pl.BlockSpec((1, _BMP, H1), lambda i: (0, i, 0)),
            pl.BlockSpec((1, _BMP, H1), lambda i: (1, i, 0)),
            pl.BlockSpec((_BMP, H1), lambda i: (i, 0)),
            pl.BlockSpec((_BMP, 1), lambda i: (i, 0)),
            pl.BlockSpec((H1, 2 * H2), lambda i: (0, 0)),
            pl.BlockSpec((1, H1), lambda i: (0, 0)),
        ],
        out_specs=pl.BlockSpec((_BMP, 2 * H2), lambda i: (i, 0)),
        out_shape=jax.ShapeDtypeStruct((NP, 2 * H2), jnp.float32),
    )(agg1, agg1, h1s, dinv, W23, b1r)


def _fin_body(q0_ref, q1_ref, gs_ref, dinv_ref, b23_ref, mu_ref, lv_ref):
    m = (dinv_ref[...] * (q0_ref[0] + q1_ref[0] + gs_ref[...])
         + b23_ref[...])
    mu_ref[...] = m[:, :H2]
    lv_ref[...] = m[:, H2:]


def _tc_fin(agg2, gs, dinv, b23r):
    return pl.pallas_call(
        _fin_body,
        grid=(N // _BM,),
        in_specs=[
            pl.BlockSpec((1, _BM, 2 * H2), lambda i: (0, i, 0)),
            pl.BlockSpec((1, _BM, 2 * H2), lambda i: (1, i, 0)),
            pl.BlockSpec((_BM, 2 * H2), lambda i: (i, 0)),
            pl.BlockSpec((_BM, 1), lambda i: (i, 0)),
            pl.BlockSpec((1, 2 * H2), lambda i: (0, 0)),
        ],
        out_specs=[
            pl.BlockSpec((_BM, H2), lambda i: (i, 0)),
            pl.BlockSpec((_BM, H2), lambda i: (i, 0)),
        ],
        out_shape=[
            jax.ShapeDtypeStruct((N, H2), jnp.float32),
            jax.ShapeDtypeStruct((N, H2), jnp.float32),
        ],
    )(agg2, agg2, gs, dinv, b23r)


_BMA = 200


def _adj_body(a_ref, bt_ref, out_ref):
    z = jnp.dot(a_ref[...], bt_ref[...], preferred_element_type=jnp.float32)
    out_ref[...] = 0.5 * jnp.tanh(0.5 * z) + 0.5


def _tc_adj(mu, muT):
    return pl.pallas_call(
        _adj_body,
        grid=(N // _BMA,),
        in_specs=[
            pl.BlockSpec((_BMA, H2), lambda i: (i, 0)),
            pl.BlockSpec((H2, N), lambda i: (0, 0)),
        ],
        out_specs=pl.BlockSpec((_BMA, N), lambda i: (i, 0)),
        out_shape=jax.ShapeDtypeStruct((N, N), jnp.float32),
    )(mu, muT)


def kernel(x, edge_index, W1, b1, W2, b2, W3, b3):
    src = edge_index[0]
    dst = edge_index[1]
    pad = EPAD - E
    pad_idx = (N + (jnp.arange(pad, dtype=jnp.int32) % (NP - N))).astype(jnp.int32)
    src_t = jnp.concatenate([src, pad_idx]).reshape(NW, CHUNKS, CH)
    dst_t = jnp.concatenate([dst, pad_idx]).reshape(NW, CHUNKS, CH)

    zeros16 = jnp.zeros((NP, 16), jnp.float32)
    zeros32 = jnp.zeros((NP, H1), jnp.float32)
    ones_blk = jnp.ones((CH, 16), jnp.float32)
    x_p = jnp.pad(x, ((0, NP - N), (0, 0)))

    degp = _sc_degree(dst_t, zeros16, ones_blk)
    h1 = _tc_mm1(x_p, W1)
    h1s, dinv = _tc_scale(degp, h1)

    agg1 = _sc_push(src_t, dst_t, h1s, zeros32)

    W23 = jnp.concatenate([W2, W3], axis=1)
    gs = _tc_mid(agg1, h1s, dinv, W23, b1.reshape(1, H1))

    agg2 = _sc_push(src_t, dst_t, gs, zeros32)

    b23 = jnp.concatenate([b2, b3]).reshape(1, 2 * H2)
    mu, logvar = _tc_fin(agg2, gs, dinv, b23)

    adj = _tc_adj(mu, mu.T)
    return (adj, mu, logvar)


# speedup vs baseline: 38.2211x; 1.0085x over previous
import functools

import jax
import jax.numpy as jnp
from jax import lax
from jax.experimental import pallas as pl
from jax.experimental.pallas import tpu as pltpu
from jax.experimental.pallas import tpu_sc as plsc

N = 10000
E = 320000
D = 128
H1 = 32
H2 = 16

NC = 2
NS = 16
NW = NC * NS
CH = 128
NBUF = 8
CHUNKS = -(-(-(-E // (NW * CH))) // NBUF) * NBUF
EPAD = NW * CHUNKS * CH
NP = -(-N // 128) * 128
RPT = NP // NS

_mesh = plsc.VectorSubcoreMesh(core_axis_name="c", subcore_axis_name="s")
_sc_params = pltpu.CompilerParams(use_tc_tiling_on_sc=False)


_DGRP = 16


@functools.partial(
    pl.kernel,
    out_type=jax.ShapeDtypeStruct((NC, NP, 16), jnp.float32),
    mesh=_mesh,
    scratch_types=[
        pltpu.VMEM_SHARED((NP, 16), jnp.float32),
        pltpu.VMEM((CHUNKS, CH), jnp.int32),
        pltpu.VMEM((CH, 16), jnp.float32),
        pltpu.SemaphoreType.DMA,
    ],
    compiler_params=_sc_params,
)
def _sc_degree(dst_hbm, zeros_hbm, ones_hbm, out_hbm, acc_sh, didx, ones_v, sem):
    c = lax.axis_index("c")
    s = lax.axis_index("s")
    wid = s * NC + c
    r0 = s * RPT
    pltpu.sync_copy(zeros_hbm.at[pl.ds(r0, RPT)], acc_sh.at[pl.ds(r0, RPT)])
    pltpu.sync_copy(dst_hbm.at[wid], didx)
    pltpu.sync_copy(ones_hbm, ones_v)
    plsc.subcore_barrier()

    def wave(g, carry):
        for b in range(_DGRP):
            pltpu.async_copy(ones_v, acc_sh.at[didx.at[g * _DGRP + b]], sem,
                             add=True)
        for b in range(_DGRP):
            pltpu.make_async_copy(ones_v, acc_sh.at[didx.at[g * _DGRP + b]],
                                  sem).wait()
        return carry

    lax.fori_loop(0, CHUNKS // _DGRP, wave, 0)
    plsc.subcore_barrier()
    pltpu.sync_copy(acc_sh.at[pl.ds(r0, RPT)], out_hbm.at[c, pl.ds(r0, RPT)])


@functools.partial(
    pl.kernel,
    out_type=jax.ShapeDtypeStruct((NC, NP, H1), jnp.float32),
    mesh=_mesh,
    scratch_types=[
        pltpu.VMEM_SHARED((NP, H1), jnp.float32),
        pltpu.VMEM((CHUNKS, CH), jnp.int32),
        pltpu.VMEM((CHUNKS, CH), jnp.int32),
        *[pltpu.VMEM((CH, H1), jnp.float32) for _ in range(NBUF)],
        *[pltpu.SemaphoreType.DMA for _ in range(2 * NBUF)],
    ],
    compiler_params=_sc_params,
)
def _sc_push(src_hbm, dst_hbm, h_hbm, zeros_hbm, out_hbm,
             acc_sh, sidx, didx, *bufs_and_sems):
    rows = bufs_and_sems[:NBUF]
    gsem = bufs_and_sems[NBUF:2 * NBUF]
    ssem = bufs_and_sems[2 * NBUF:]
    c = lax.axis_index("c")
    s = lax.axis_index("s")
    wid = s * NC + c
    r0 = s * RPT
    pltpu.sync_copy(zeros_hbm.at[pl.ds(r0, RPT)], acc_sh.at[pl.ds(r0, RPT)])
    pltpu.sync_copy(src_hbm.at[wid], sidx)
    pltpu.sync_copy(dst_hbm.at[wid], didx)
    plsc.subcore_barrier()

    for b in range(NBUF):
        pltpu.async_copy(h_hbm.at[sidx.at[b]], rows[b], gsem[b])

    def group(g, carry):
        for b in range(NBUF):
            j = g * NBUF + b
            pltpu.make_async_copy(h_hbm.at[sidx.at[j]], rows[b], gsem[b]).wait()
            pltpu.async_copy(rows[b], acc_sh.at[didx.at[j]], ssem[b], add=True)
            jn = j + NBUF

            @pl.when(jn < CHUNKS)
            def _():
                pltpu.make_async_copy(rows[b], acc_sh.at[didx.at[j]],
                                      ssem[b]).wait()
                pltpu.async_copy(h_hbm.at[sidx.at[jn]], rows[b], gsem[b])

        return carry

    lax.fori_loop(0, CHUNKS // NBUF, group, 0)
    for b in range(NBUF):
        pltpu.make_async_copy(rows[b], acc_sh.at[didx.at[0]], ssem[b]).wait()
    plsc.subcore_barrier()
    pltpu.sync_copy(acc_sh.at[pl.ds(r0, RPT)], out_hbm.at[c, pl.ds(r0, RPT)])


_BMP = NP // 8
_BM = 2000


def _mm1_body(x_ref, w1_ref, h1_ref):
    h1_ref[...] = jnp.dot(x_ref[...], w1_ref[...],
                          preferred_element_type=jnp.float32)


def _tc_mm1(x_p, W1):
    return pl.pallas_call(
        _mm1_body,
        grid=(NP // _BMP,),
        in_specs=[
            pl.BlockSpec((_BMP, D), lambda i: (i, 0)),
            pl.BlockSpec((D, H1), lambda i: (0, 0)),
        ],
        out_specs=pl.BlockSpec((_BMP, H1), lambda i: (i, 0)),
        out_shape=jax.ShapeDtypeStruct((NP, H1), jnp.float32),
    )(x_p, W1)


def _scale_body(d0_ref, d1_ref, h1_ref, h1s_ref, dinv_ref):
    deg = 1.0 + d0_ref[0, :, :1] + d1_ref[0, :, :1]
    dinv = lax.rsqrt(deg)
    h1s_ref[...] = h1_ref[...] * dinv
    dinv_ref[...] = dinv


def _tc_scale(degp, h1):
    return pl.pallas_call(
        _scale_body,
        grid=(NP // _BMP,),
        in_specs=[
            pl.BlockSpec((1, _BMP, 16), lambda i: (0, i, 0)),
            pl.BlockSpec((1, _BMP, 16), lambda i: (1, i, 0)),
            pl.BlockSpec((_BMP, H1), lambda i: (i, 0)),
        ],
        out_specs=[
            pl.BlockSpec((_BMP, H1), lambda i: (i, 0)),
            pl.BlockSpec((_BMP, 1), lambda i: (i, 0)),
        ],
        out_shape=[
            jax.ShapeDtypeStruct((NP, H1), jnp.float32),
            jax.ShapeDtypeStruct((NP, 1), jnp.float32),
        ],
    )(degp, degp, h1)


def _mid_body(p0_ref, p1_ref, h1s_ref, dinv_ref, w23_ref, b1_ref, gs_ref):
    dinv = dinv_ref[...]
    hidden1 = dinv * (p0_ref[0] + p1_ref[0] + h1s_ref[...]) + b1_ref[...]
    g = jnp.dot(hidden1, w23_ref[...], preferred_element_type=jnp.float32)
    i = pl.program_id(0)
    row = i * _BMP + lax.broadcasted_iota(jnp.int32, (_BMP, 2 * H2), 0)
    gs_ref[...] = jnp.where(row < N, dinv * g, 0.0)


def _tc_mid(agg1, h1s, dinv, W23, b1r):
    return pl.pallas_call(
        _mid_body,
        grid=(NP // _BMP,),
        in_specs=[
            pl.BlockSpec((1, _BMP, H1), lambda i: (0, i, 0)),
            pl.BlockSpec((1, _BMP, H1), lambda i: (1, i, 0)),
            pl.BlockSpec((_BMP, H1), lambda i: (i, 0)),
            pl.BlockSpec((_BMP, 1), lambda i: (i, 0)),
            pl.BlockSpec((H1, 2 * H2), lambda i: (0, 0)),
            pl.BlockSpec((1, H1), lambda i: (0, 0)),
        ],
        out_specs=pl.BlockSpec((_BMP, 2 * H2), lambda i: (i, 0)),
        out_shape=jax.ShapeDtypeStruct((NP, 2 * H2), jnp.float32),
    )(agg1, agg1, h1s, dinv, W23, b1r)


_BMA = 200
_PHA = N // _BM
_PHB = N // _BMA


def _dec_body(q0_ref, q1_ref, gs_ref, dinv_ref, b23_ref,
              mu_ref, lv_ref, adj_ref, scr_ref):
    i = pl.program_id(0)

    @pl.when(i < _PHA)
    def _finalize():
        m = (dinv_ref[...] * (q0_ref[0] + q1_ref[0] + gs_ref[...])
             + b23_ref[...])
        mu = m[:, :H2]
        mu_ref[...] = mu
        lv_ref[...] = m[:, H2:]
        scr_ref[pl.ds(i * _BM, _BM), :] = mu

    @pl.when(i >= _PHA)
    def _decode():
        r = (i - _PHA) * _BMA
        a = scr_ref[pl.ds(r, _BMA), :]
        z = lax.dot_general(a, scr_ref[...], (((1,), (1,)), ((), ())),
                            preferred_element_type=jnp.float32)
        adj_ref[...] = 0.5 * jnp.tanh(0.5 * z) + 0.5


def _tc_dec(agg2, gs, dinv, b23r):
    clampA = lambda i: (jnp.minimum(i, _PHA - 1), 0)
    clampA3 = lambda c: lambda i: (c, jnp.minimum(i, _PHA - 1), 0)
    return pl.pallas_call(
        _dec_body,
        grid=(_PHA + _PHB,),
        in_specs=[
            pl.BlockSpec((1, _BM, 2 * H2), clampA3(0)),
            pl.BlockSpec((1, _BM, 2 * H2), clampA3(1)),
            pl.BlockSpec((_BM, 2 * H2), clampA),
            pl.BlockSpec((_BM, 1), clampA),
            pl.BlockSpec((1, 2 * H2), lambda i: (0, 0)),
        ],
        out_specs=[
            pl.BlockSpec((_BM, H2), clampA),
            pl.BlockSpec((_BM, H2), clampA),
            pl.BlockSpec((_BMA, N), lambda i: (jnp.maximum(i - _PHA, 0), 0)),
        ],
        out_shape=[
            jax.ShapeDtypeStruct((N, H2), jnp.float32),
            jax.ShapeDtypeStruct((N, H2), jnp.float32),
            jax.ShapeDtypeStruct((N, N), jnp.float32),
        ],
        scratch_shapes=[pltpu.VMEM((N, H2), jnp.float32)],
    )(agg2, agg2, gs, dinv, b23r)


def kernel(x, edge_index, W1, b1, W2, b2, W3, b3):
    src = edge_index[0]
    dst = edge_index[1]
    pad = EPAD - E
    pad_idx = (N + (jnp.arange(pad, dtype=jnp.int32) % (NP - N))).astype(jnp.int32)
    src_t = jnp.concatenate([src, pad_idx]).reshape(NW, CHUNKS, CH)
    dst_t = jnp.concatenate([dst, pad_idx]).reshape(NW, CHUNKS, CH)

    zeros16 = jnp.zeros((NP, 16), jnp.float32)
    zeros32 = jnp.zeros((NP, H1), jnp.float32)
    ones_blk = jnp.ones((CH, 16), jnp.float32)
    x_p = jnp.pad(x, ((0, NP - N), (0, 0)))

    degp = _sc_degree(dst_t, zeros16, ones_blk)
    h1 = _tc_mm1(x_p, W1)
    h1s, dinv = _tc_scale(degp, h1)

    agg1 = _sc_push(src_t, dst_t, h1s, zeros32)

    W23 = jnp.concatenate([W2, W3], axis=1)
    gs = _tc_mid(agg1, h1s, dinv, W23, b1.reshape(1, H1))

    agg2 = _sc_push(src_t, dst_t, gs, zeros32)

    b23 = jnp.concatenate([b2, b3]).reshape(1, 2 * H2)
    mu, logvar, adj = _tc_dec(agg2, gs, dinv, b23)
    return (adj, mu, logvar)


# speedup vs baseline: 42.2081x; 1.1043x over previous
import functools

import jax
import jax.numpy as jnp
from jax import lax
from jax.experimental import pallas as pl
from jax.experimental.pallas import tpu as pltpu
from jax.experimental.pallas import tpu_sc as plsc

N = 10000
E = 320000
D = 128
H1 = 32
H2 = 16

NC = 2
NS = 16
NW = NC * NS
CH = 128
NBUF = 8
CHUNKS = -(-(-(-E // (NW * CH))) // NBUF) * NBUF
EPAD = NW * CHUNKS * CH
NP = -(-N // 128) * 128
RPT = NP // NS

_mesh = plsc.VectorSubcoreMesh(core_axis_name="c", subcore_axis_name="s")
_sc_params = pltpu.CompilerParams(use_tc_tiling_on_sc=False)


_DGRP = 16


@functools.partial(
    pl.kernel,
    out_type=jax.ShapeDtypeStruct((NC, NP, H1), jnp.float32),
    mesh=_mesh,
    scratch_types=[
        pltpu.VMEM_SHARED((NP, H1), jnp.float32),
        pltpu.VMEM((CHUNKS, CH), jnp.int32),
        pltpu.VMEM((CH, H1), jnp.float32),
        pltpu.SemaphoreType.DMA,
    ],
    compiler_params=_sc_params,
)
def _sc_degree(dst_hbm, zeros_hbm, ones_hbm, out_hbm, acc_sh, didx, ones_v, sem):
    c = lax.axis_index("c")
    s = lax.axis_index("s")
    wid = s * NC + c
    r0 = s * RPT
    pltpu.sync_copy(zeros_hbm.at[pl.ds(r0, RPT)], acc_sh.at[pl.ds(r0, RPT)])
    pltpu.sync_copy(dst_hbm.at[wid], didx)
    pltpu.sync_copy(ones_hbm, ones_v)
    plsc.subcore_barrier()

    def wave(g, carry):
        for b in range(_DGRP):
            pltpu.async_copy(ones_v, acc_sh.at[didx.at[g * _DGRP + b]], sem,
                             add=True)
        for b in range(_DGRP):
            pltpu.make_async_copy(ones_v, acc_sh.at[didx.at[g * _DGRP + b]],
                                  sem).wait()
        return carry

    lax.fori_loop(0, CHUNKS // _DGRP, wave, 0)
    plsc.subcore_barrier()
    pltpu.sync_copy(acc_sh.at[pl.ds(r0, RPT)], out_hbm.at[c, pl.ds(r0, RPT)])


@functools.partial(
    pl.kernel,
    out_type=jax.ShapeDtypeStruct((NC, NP, H1), jnp.float32),
    mesh=_mesh,
    scratch_types=[
        pltpu.VMEM_SHARED((NP, H1), jnp.float32),
        pltpu.VMEM((CHUNKS, CH), jnp.int32),
        pltpu.VMEM((CHUNKS, CH), jnp.int32),
        *[pltpu.VMEM((CH, H1), jnp.float32) for _ in range(NBUF)],
        *[pltpu.SemaphoreType.DMA for _ in range(2 * NBUF)],
    ],
    compiler_params=_sc_params,
)
def _sc_push(src_hbm, dst_hbm, h_hbm, zeros_hbm, out_hbm,
             acc_sh, sidx, didx, *bufs_and_sems):
    rows = bufs_and_sems[:NBUF]
    gsem = bufs_and_sems[NBUF:2 * NBUF]
    ssem = bufs_and_sems[2 * NBUF:]
    c = lax.axis_index("c")
    s = lax.axis_index("s")
    wid = s * NC + c
    r0 = s * RPT
    pltpu.sync_copy(zeros_hbm.at[pl.ds(r0, RPT)], acc_sh.at[pl.ds(r0, RPT)])
    pltpu.sync_copy(src_hbm.at[wid], sidx)
    pltpu.sync_copy(dst_hbm.at[wid], didx)
    plsc.subcore_barrier()

    for b in range(NBUF):
        pltpu.async_copy(h_hbm.at[sidx.at[b]], rows[b], gsem[b])

    def group(g, carry):
        for b in range(NBUF):
            j = g * NBUF + b
            pltpu.make_async_copy(h_hbm.at[sidx.at[j]], rows[b], gsem[b]).wait()
            pltpu.async_copy(rows[b], acc_sh.at[didx.at[j]], ssem[b], add=True)
            jn = j + NBUF

            @pl.when(jn < CHUNKS)
            def _():
                pltpu.make_async_copy(rows[b], acc_sh.at[didx.at[j]],
                                      ssem[b]).wait()
                pltpu.async_copy(h_hbm.at[sidx.at[jn]], rows[b], gsem[b])

        return carry

    lax.fori_loop(0, CHUNKS // NBUF, group, 0)
    for b in range(NBUF):
        pltpu.make_async_copy(rows[b], acc_sh.at[didx.at[0]], ssem[b]).wait()
    plsc.subcore_barrier()
    pltpu.sync_copy(acc_sh.at[pl.ds(r0, RPT)], out_hbm.at[c, pl.ds(r0, RPT)])


NPK = NP * H1 // 128
NK = N * H1 // 128
_BMK = NPK // 4


def _blockdiag4(W):
    z = jnp.zeros_like(W)
    r1 = jnp.concatenate([W, z, z, z], axis=1)
    r2 = jnp.concatenate([z, W, z, z], axis=1)
    r3 = jnp.concatenate([z, z, W, z], axis=1)
    r4 = jnp.concatenate([z, z, z, W], axis=1)
    return jnp.concatenate([r1, r2, r3, r4], axis=0)


def _mm1_body(x4_ref, w1b_ref, h1_ref):
    h1_ref[...] = jnp.dot(x4_ref[...], w1b_ref[...],
                          preferred_element_type=jnp.float32)


def _tc_mm1(x4, W1b):
    return pl.pallas_call(
        _mm1_body,
        grid=(NPK // _BMK,),
        in_specs=[
            pl.BlockSpec((_BMK, 4 * D), lambda i: (i, 0)),
            pl.BlockSpec((4 * D, 128), lambda i: (0, 0)),
        ],
        out_specs=pl.BlockSpec((_BMK, 128), lambda i: (i, 0)),
        out_shape=jax.ShapeDtypeStruct((NPK, 128), jnp.float32),
    )(x4, W1b)


def _scale_body(d0_ref, d1_ref, h1_ref, h1s_ref, dinv_ref):
    dinv = lax.rsqrt(1.0 + d0_ref[0] + d1_ref[0])
    h1s_ref[...] = h1_ref[...] * dinv
    dinv_ref[...] = dinv


def _tc_scale(degp4, h1p):
    return pl.pallas_call(
        _scale_body,
        grid=(NPK // _BMK,),
        in_specs=[
            pl.BlockSpec((1, _BMK, 128), lambda i: (0, i, 0)),
            pl.BlockSpec((1, _BMK, 128), lambda i: (1, i, 0)),
            pl.BlockSpec((_BMK, 128), lambda i: (i, 0)),
        ],
        out_specs=[
            pl.BlockSpec((_BMK, 128), lambda i: (i, 0)),
            pl.BlockSpec((_BMK, 128), lambda i: (i, 0)),
        ],
        out_shape=[
            jax.ShapeDtypeStruct((NPK, 128), jnp.float32),
            jax.ShapeDtypeStruct((NPK, 128), jnp.float32),
        ],
    )(degp4, degp4, h1p)


def _mid_body(p0_ref, p1_ref, h1s_ref, dinv_ref, w23b_ref, b1_ref, gs_ref):
    dinv = dinv_ref[...]
    hidden1 = dinv * (p0_ref[0] + p1_ref[0] + h1s_ref[...]) + b1_ref[...]
    g = jnp.dot(hidden1, w23b_ref[...], preferred_element_type=jnp.float32)
    i = pl.program_id(0)
    row = i * _BMK + lax.broadcasted_iota(jnp.int32, (_BMK, 128), 0)
    gs_ref[...] = jnp.where(row < NK, dinv * g, 0.0)


def _tc_mid(agg1p, h1sp, dinvp, W23b, b1p):
    return pl.pallas_call(
        _mid_body,
        grid=(NPK // _BMK,),
        in_specs=[
            pl.BlockSpec((1, _BMK, 128), lambda i: (0, i, 0)),
            pl.BlockSpec((1, _BMK, 128), lambda i: (1, i, 0)),
            pl.BlockSpec((_BMK, 128), lambda i: (i, 0)),
            pl.BlockSpec((_BMK, 128), lambda i: (i, 0)),
            pl.BlockSpec((128, 128), lambda i: (0, 0)),
            pl.BlockSpec((1, 128), lambda i: (0, 0)),
        ],
        out_specs=pl.BlockSpec((_BMK, 128), lambda i: (i, 0)),
        out_shape=jax.ShapeDtypeStruct((NPK, 128), jnp.float32),
    )(agg1p, agg1p, h1sp, dinvp, W23b, b1p)


_BMA = 200
_BM = 2000
_PHA = N // _BM
_PHB = N // _BMA


def _dec_body(q0_ref, q1_ref, gs_ref, dinv_ref, b23_ref,
              mu_ref, lv_ref, adj_ref, scr_ref):
    i = pl.program_id(0)

    @pl.when(i < _PHA)
    def _finalize():
        m = (dinv_ref[:, :1] * (q0_ref[0] + q1_ref[0] + gs_ref[...])
             + b23_ref[...])
        mu = m[:, :H2]
        mu_ref[...] = mu
        lv_ref[...] = m[:, H2:]
        scr_ref[pl.ds(i * _BM, _BM), :] = mu

    @pl.when(i >= _PHA)
    def _decode():
        r = (i - _PHA) * _BMA
        a = scr_ref[pl.ds(r, _BMA), :]
        z = lax.dot_general(a, scr_ref[...], (((1,), (1,)), ((), ())),
                            preferred_element_type=jnp.float32)
        adj_ref[...] = 0.5 * jnp.tanh(0.5 * z) + 0.5


def _tc_dec(agg2, gs_nm, dinv_nm, b23r):
    clampA = lambda i: (jnp.minimum(i, _PHA - 1), 0)
    clampA3 = lambda c: lambda i: (c, jnp.minimum(i, _PHA - 1), 0)
    return pl.pallas_call(
        _dec_body,
        grid=(_PHA + _PHB,),
        in_specs=[
            pl.BlockSpec((1, _BM, H1), clampA3(0)),
            pl.BlockSpec((1, _BM, H1), clampA3(1)),
            pl.BlockSpec((_BM, H1), clampA),
            pl.BlockSpec((_BM, H1), clampA),
            pl.BlockSpec((1, H1), lambda i: (0, 0)),
        ],
        out_specs=[
            pl.BlockSpec((_BM, H2), clampA),
            pl.BlockSpec((_BM, H2), clampA),
            pl.BlockSpec((_BMA, N), lambda i: (jnp.maximum(i - _PHA, 0), 0)),
        ],
        out_shape=[
            jax.ShapeDtypeStruct((N, H2), jnp.float32),
            jax.ShapeDtypeStruct((N, H2), jnp.float32),
            jax.ShapeDtypeStruct((N, N), jnp.float32),
        ],
        scratch_shapes=[pltpu.VMEM((N, H2), jnp.float32)],
    )(agg2, agg2, gs_nm, dinv_nm, b23r)


def kernel(x, edge_index, W1, b1, W2, b2, W3, b3):
    src = edge_index[0]
    dst = edge_index[1]
    pad = EPAD - E
    pad_idx = (N + (jnp.arange(pad, dtype=jnp.int32) % (NP - N))).astype(jnp.int32)
    src_t = jnp.concatenate([src, pad_idx]).reshape(NW, CHUNKS, CH)
    dst_t = jnp.concatenate([dst, pad_idx]).reshape(NW, CHUNKS, CH)

    zeros32 = jnp.zeros((NP, H1), jnp.float32)
    ones_blk = jnp.ones((CH, H1), jnp.float32)
    x_p = jnp.pad(x, ((0, NP - N), (0, 0)))
    x4 = x_p.reshape(NPK, 4 * D)

    degp = _sc_degree(dst_t, zeros32, ones_blk)
    W1b = _blockdiag4(W1)
    h1p = _tc_mm1(x4, W1b)
    h1sp, dinvp = _tc_scale(degp.reshape(NC, NPK, 128), h1p)

    agg1 = _sc_push(src_t, dst_t, h1sp.reshape(NP, H1), zeros32)

    W23b = _blockdiag4(jnp.concatenate([W2, W3], axis=1))
    b1p = jnp.tile(b1, 4).reshape(1, 128)
    gsp = _tc_mid(agg1.reshape(NC, NPK, 128), h1sp, dinvp, W23b, b1p)

    agg2 = _sc_push(src_t, dst_t, gsp.reshape(NP, H1), zeros32)

    b23 = jnp.concatenate([b2, b3]).reshape(1, H1)
    mu, logvar, adj = _tc_dec(agg2, gsp.reshape(NP, H1),
                              dinvp.reshape(NP, H1), b23)
    return (adj, mu, logvar)
